# baseline (speedup 1.0000x reference)
import functools

import jax
import jax.numpy as jnp
from jax.experimental import pallas as pl
from jax.experimental.pallas import tpu as pltpu

_CHUNK = 512


def _weights_all(c0, c1, lengths, L):
    B = c0.shape[0]
    idx = jax.lax.broadcasted_iota(jnp.int32, (B, L), 1)
    valid = idx < lengths
    same = (c0 == pltpu.roll(c0, 1, axis=1)) & (c1 == pltpu.roll(c1, 1, axis=1))
    ng = ((idx == 0) | jnp.logical_not(same)) & valid

    s = jnp.where(ng, idx, -1)
    k = 1
    while k < L:
        s = jnp.maximum(s, jnp.where(idx >= k, pltpu.roll(s, k, axis=1), -1))
        k *= 2
    t = jnp.where(ng, idx, L)
    t = jnp.where(idx < L - 1, pltpu.roll(t, L - 1, axis=1), L)
    k = 1
    while k < L:
        t = jnp.minimum(t, jnp.where(idx < L - k, pltpu.roll(t, L - k, axis=1), L))
        k *= 2

    run_len = (jnp.minimum(t, lengths) - s).astype(jnp.float32)
    num_groups = jnp.sum(ng.astype(jnp.float32), axis=1, keepdims=True)
    denom = num_groups * run_len
    safe = valid & (denom > 0.0)
    return jnp.where(safe, 1.0 / jnp.where(safe, denom, 1.0), 0.0)


def _pool_kernel(len_ref, vq_ref, feat_hbm, out_ref, w_ref, b_of, c_of, *,
                 B, N, L, D, chunk):
    c0 = vq_ref[:, 0, :]
    c1 = vq_ref[:, 1, :]
    lengths = jnp.concatenate(
        [jnp.full((1, 1), len_ref[i], jnp.int32) for i in range(B)], axis=0)
    w_ref[...] = _weights_all(c0, c1, lengths, L)
    out_ref[...] = jnp.zeros_like(out_ref)

    def n_chunks(b):
        return (len_ref[b] + chunk - 1) // chunk

    total = n_chunks(0)
    for i in range(1, B):
        total = total + n_chunks(i)

    def build(j, carry):
        b, c = carry
        b_of[j] = b
        c_of[j] = c
        last = (c + 1) == n_chunks(b)
        return (jnp.where(last, b + 1, b), jnp.where(last, 0, c + 1))

    jax.lax.fori_loop(0, total, build, (jnp.int32(0), jnp.int32(0)))

    def inner(idxs, feat_chunk):
        j = idxs[0]
        b = b_of[j]
        c = c_of[j]
        w_chunk = w_ref[pl.ds(b, 1), pl.ds(c * chunk, chunk)]
        out_ref[pl.ds(b, 1), 0] += jnp.dot(w_chunk, feat_chunk[0, 0],
                                           preferred_element_type=jnp.float32)

    pipe = pltpu.emit_pipeline(
        inner,
        grid=(total,),
        in_specs=[pl.BlockSpec((1, 1, chunk, D),
                               lambda j: (b_of[j], N - 1, c_of[j], 0),
                               pipeline_mode=pl.Buffered(buffer_count=6))],
        _explicit_indices=True,
    )
    pipe(feat_hbm)


@jax.jit
def kernel(input_feature, input_lengths, vq_indices):
    B, N, L, D = input_feature.shape
    lengths = input_lengths.astype(jnp.int32)
    vq_t = jnp.transpose(vq_indices.astype(jnp.int32), (0, 2, 1))

    max_steps = B * (L // _CHUNK)
    grid_spec = pltpu.PrefetchScalarGridSpec(
        num_scalar_prefetch=1,
        grid=(1,),
        in_specs=[
            pl.BlockSpec((B, 2, L), lambda g, lens: (0, 0, 0)),
            pl.BlockSpec(memory_space=pl.ANY),
        ],
        out_specs=pl.BlockSpec((B, 1, D), lambda g, lens: (0, 0, 0)),
        scratch_shapes=[
            pltpu.VMEM((B, L), jnp.float32),
            pltpu.SMEM((max_steps,), jnp.int32),
            pltpu.SMEM((max_steps,), jnp.int32),
        ],
    )
    out = pl.pallas_call(
        functools.partial(_pool_kernel, B=B, N=N, L=L, D=D, chunk=_CHUNK),
        grid_spec=grid_spec,
        out_shape=jax.ShapeDtypeStruct((B, 1, D), jnp.float32),
    )(lengths, vq_t, input_feature)
    return out[:, 0, :]


# speedup vs baseline: 1.0616x; 1.0616x over previous
import functools

import jax
import jax.numpy as jnp
from jax.experimental import pallas as pl
from jax.experimental.pallas import tpu as pltpu

_CHUNK = 512


def _weights_all(c0, c1, lengths, L):
    B = c0.shape[0]
    idx = jax.lax.broadcasted_iota(jnp.int32, (B, L), 1)
    valid = idx < lengths
    same = (c0 == pltpu.roll(c0, 1, axis=1)) & (c1 == pltpu.roll(c1, 1, axis=1))
    ng = ((idx == 0) | jnp.logical_not(same)) & valid

    s = jnp.where(ng, idx, -1)
    k = 1
    while k < L:
        s = jnp.maximum(s, jnp.where(idx >= k, pltpu.roll(s, k, axis=1), -1))
        k *= 2
    t = jnp.where(ng, idx, L)
    t = jnp.where(idx < L - 1, pltpu.roll(t, L - 1, axis=1), L)
    k = 1
    while k < L:
        t = jnp.minimum(t, jnp.where(idx < L - k, pltpu.roll(t, L - k, axis=1), L))
        k *= 2

    run_len = (jnp.minimum(t, lengths) - s).astype(jnp.float32)
    num_groups = jnp.sum(ng.astype(jnp.float32), axis=1, keepdims=True)
    denom = num_groups * run_len
    safe = valid & (denom > 0.0)
    return jnp.where(safe, 1.0 / jnp.where(safe, denom, 1.0), 0.0)


def _pool_kernel(len_ref, vq_ref, feat_hbm, out_ref, w_ref, b_of, c_of, *,
                 B, N, L, D, chunk):
    def n_chunks(b):
        return (len_ref[b] + chunk - 1) // chunk

    total = n_chunks(0)
    for i in range(1, B):
        total = total + n_chunks(i)

    def build(j, carry):
        b, c = carry
        b_of[j] = b
        c_of[j] = c
        last = (c + 1) == n_chunks(b)
        return (jnp.where(last, b + 1, b), jnp.where(last, 0, c + 1))

    jax.lax.fori_loop(0, total, build, (jnp.int32(0), jnp.int32(0)))

    def inner(idxs, feat_chunk):
        j = idxs[0]

        @pl.when(j == 0)
        def _():
            c0 = vq_ref[:, 0, :]
            c1 = vq_ref[:, 1, :]
            lengths = jnp.concatenate(
                [jnp.full((1, 1), len_ref[i], jnp.int32) for i in range(B)],
                axis=0)
            w_ref[...] = _weights_all(c0, c1, lengths, L)
            out_ref[...] = jnp.zeros_like(out_ref)

        b = b_of[j]
        c = c_of[j]
        w_chunk = w_ref[pl.ds(b, 1), pl.ds(c * chunk, chunk)]
        out_ref[pl.ds(b, 1), 0] += jnp.dot(w_chunk, feat_chunk[0, 0],
                                           preferred_element_type=jnp.float32)

    pipe = pltpu.emit_pipeline(
        inner,
        grid=(total,),
        in_specs=[pl.BlockSpec((1, 1, chunk, D),
                               lambda j: (b_of[j], N - 1, c_of[j], 0),
                               pipeline_mode=pl.Buffered(buffer_count=4))],
        _explicit_indices=True,
    )
    pipe(feat_hbm)


@jax.jit
def kernel(input_feature, input_lengths, vq_indices):
    B, N, L, D = input_feature.shape
    lengths = input_lengths.astype(jnp.int32)
    vq_t = jnp.transpose(vq_indices.astype(jnp.int32), (0, 2, 1))

    max_steps = B * (L // _CHUNK)
    grid_spec = pltpu.PrefetchScalarGridSpec(
        num_scalar_prefetch=1,
        grid=(1,),
        in_specs=[
            pl.BlockSpec((B, 2, L), lambda g, lens: (0, 0, 0)),
            pl.BlockSpec(memory_space=pl.ANY),
        ],
        out_specs=pl.BlockSpec((B, 1, D), lambda g, lens: (0, 0, 0)),
        scratch_shapes=[
            pltpu.VMEM((B, L), jnp.float32),
            pltpu.SMEM((max_steps,), jnp.int32),
            pltpu.SMEM((max_steps,), jnp.int32),
        ],
    )
    out = pl.pallas_call(
        functools.partial(_pool_kernel, B=B, N=N, L=L, D=D, chunk=_CHUNK),
        grid_spec=grid_spec,
        out_shape=jax.ShapeDtypeStruct((B, 1, D), jnp.float32),
    )(lengths, vq_t, input_feature)
    return out[:, 0, :]
